# Initial kernel scaffold; baseline (speedup 1.0000x reference)
import jax
import jax.numpy as jnp
from jax.experimental import pallas as pl


def kernel(sequence, segment_label, token_table, seg_table, gamma, beta, pe):
    raise NotImplementedError("write your pallas kernel here")


# speedup vs baseline: 3.0244x; 3.0244x over previous
import functools

import jax
import jax.numpy as jnp
from jax import lax
from jax.experimental import pallas as pl
from jax.experimental.pallas import tpu as pltpu
from jax.experimental.pallas import tpu_sc as plsc

DIM = 128
NSEG = 3
EPS = 1e-5
LANES = 16
NJ = DIM // LANES
NC = 2
NS = 16
NW = NC * NS
C = 128


def _build(B, S, V):
    rows_total = B * S
    rows_per_tile = rows_total // NW
    nchunk = rows_per_tile // C
    chunks_per_seq = S // C
    inv_d = 1.0 / DIM

    mesh = plsc.VectorSubcoreMesh(core_axis_name="c", subcore_axis_name="s")

    @functools.partial(
        pl.kernel,
        mesh=mesh,
        out_type=jax.ShapeDtypeStruct((rows_total, DIM), jnp.float32),
        compiler_params=pltpu.CompilerParams(needs_layout_passes=False),
        scratch_types=[
            pltpu.VMEM((S * DIM,), jnp.float32),
            pltpu.VMEM((NSEG * DIM,), jnp.float32),
            pltpu.VMEM((DIM,), jnp.float32),
            pltpu.VMEM((DIM,), jnp.float32),
            pltpu.VMEM((C,), jnp.int32),
            pltpu.VMEM((C,), jnp.int32),
            pltpu.VMEM((C, DIM), jnp.float32),
            pltpu.SemaphoreType.DMA,
        ],
    )
    def sc_kernel(seq_hbm, lab_hbm, tok_hbm, segtab_hbm, gamma_hbm, beta_hbm,
                  pe_hbm, out_hbm, pe_v, segtab_v, gamma_v, beta_v, idx_v,
                  lab_v, buf_v, sem):
        wid = lax.axis_index("s") * NC + lax.axis_index("c")
        row_base = wid * rows_per_tile
        lane = lax.iota(jnp.int32, LANES)

        pltpu.sync_copy(pe_hbm, pe_v)
        pltpu.sync_copy(segtab_hbm, segtab_v)
        pltpu.sync_copy(gamma_hbm, gamma_v)
        pltpu.sync_copy(beta_hbm, beta_v)

        gammas = [gamma_v[pl.ds(j * LANES, LANES)] for j in range(NJ)]
        betas = [beta_v[pl.ds(j * LANES, LANES)] for j in range(NJ)]

        def chunk_body(k, _):
            base = row_base + k * C
            p0 = lax.rem(k, chunks_per_seq) * C

            pltpu.sync_copy(seq_hbm.at[pl.ds(base, C)], idx_v)
            pltpu.sync_copy(lab_hbm.at[pl.ds(base, C)], lab_v)
            pltpu.async_copy(tok_hbm.at[idx_v], buf_v, sem).wait()

            def row_body(r, _):
                rfull = jnp.full((LANES,), r, jnp.int32)
                labelb = plsc.load_gather(lab_v, [rfull])
                segbase = labelb * DIM + lane
                pbase = (p0 + r) * DIM
                s = None
                q = None
                xs = []
                for j in range(NJ):
                    t = buf_v[r, pl.ds(j * LANES, LANES)]
                    p = pe_v[pl.ds(pbase + j * LANES, LANES)]
                    sg = plsc.load_gather(segtab_v, [segbase + j * LANES])
                    x = (t + p) + sg
                    xs.append(x)
                    s = x if s is None else s + x
                    q = x * x if q is None else q + x * x
                ssum = jnp.sum(s)
                qsum = jnp.sum(q)
                mean = jnp.full((LANES,), ssum, jnp.float32) * inv_d
                var = jnp.full((LANES,), qsum, jnp.float32) * inv_d - mean * mean
                ve = var + EPS
                seed = jnp.int32(0x5F3759DF) - (plsc.bitcast(ve, jnp.int32) >> 1)
                y = plsc.bitcast(seed, jnp.float32)
                for _ in range(3):
                    y = y * (1.5 - 0.5 * ve * y * y)
                for j in range(NJ):
                    out = (xs[j] - mean) * y * gammas[j] + betas[j]
                    buf_v[r, pl.ds(j * LANES, LANES)] = out
                return 0

            lax.fori_loop(0, C, row_body, 0, unroll=2)

            pltpu.sync_copy(buf_v, out_hbm.at[pl.ds(base, C)])
            return 0

        lax.fori_loop(0, nchunk, chunk_body, 0)

    return sc_kernel


def kernel(sequence, segment_label, token_table, seg_table, gamma, beta, pe):
    B, S = sequence.shape
    V = token_table.shape[0]
    seq = sequence.reshape(-1).astype(jnp.int32)
    lab = segment_label.reshape(-1).astype(jnp.int32)
    pe_s = pe[:S].reshape(-1)
    out = _build(B, S, V)(seq, lab, token_table, seg_table.reshape(-1), gamma,
                          beta, pe_s)
    return out.reshape(B, S, DIM)


# speedup vs baseline: 3.9519x; 1.3067x over previous
import functools

import jax
import jax.numpy as jnp
from jax import lax
from jax.experimental import pallas as pl
from jax.experimental.pallas import tpu as pltpu
from jax.experimental.pallas import tpu_sc as plsc

DIM = 128
NSEG = 3
EPS = 1e-5
LANES = 16
NJ = DIM // LANES
NC = 2
NS = 16
NW = NC * NS
C = 128


def _build(B, S, V):
    rows_total = B * S
    rows_per_tile = rows_total // NW
    nchunk = rows_per_tile // C
    chunks_per_seq = S // C
    inv_d = 1.0 / DIM

    mesh = plsc.VectorSubcoreMesh(core_axis_name="c", subcore_axis_name="s")

    @functools.partial(
        pl.kernel,
        mesh=mesh,
        out_type=jax.ShapeDtypeStruct((rows_total, DIM), jnp.float32),
        compiler_params=pltpu.CompilerParams(needs_layout_passes=False),
        scratch_types=[
            pltpu.VMEM((S * DIM,), jnp.float32),
            pltpu.VMEM((NSEG * DIM,), jnp.float32),
            pltpu.VMEM((DIM,), jnp.float32),
            pltpu.VMEM((DIM,), jnp.float32),
            pltpu.VMEM((C,), jnp.int32),
            pltpu.VMEM((C,), jnp.int32),
            pltpu.VMEM((C,), jnp.int32),
            pltpu.VMEM((C,), jnp.int32),
            pltpu.VMEM((C, DIM), jnp.float32),
            pltpu.VMEM((C, DIM), jnp.float32),
            pltpu.SemaphoreType.DMA,
            pltpu.SemaphoreType.DMA,
            pltpu.SemaphoreType.DMA,
            pltpu.SemaphoreType.DMA,
            pltpu.SemaphoreType.DMA,
            pltpu.SemaphoreType.DMA,
            pltpu.SemaphoreType.DMA,
            pltpu.SemaphoreType.DMA,
        ],
    )
    def sc_kernel(seq_hbm, lab_hbm, tok_hbm, segtab_hbm, gamma_hbm, beta_hbm,
                  pe_hbm, out_hbm, pe_v, segtab_v, gamma_v, beta_v, idx0, idx1,
                  lab0, lab1, buf0, buf1, isem0, isem1, lsem0, lsem1, gsem0,
                  gsem1, osem0, osem1):
        wid = lax.axis_index("s") * NC + lax.axis_index("c")
        row_base = wid * rows_per_tile
        lane = lax.iota(jnp.int32, LANES)
        idx = [idx0, idx1]
        labs = [lab0, lab1]
        buf = [buf0, buf1]
        isem = [isem0, isem1]
        lsem = [lsem0, lsem1]
        gsem = [gsem0, gsem1]
        osem = [osem0, osem1]

        pltpu.sync_copy(pe_hbm, pe_v)
        pltpu.sync_copy(segtab_hbm, segtab_v)
        pltpu.sync_copy(gamma_hbm, gamma_v)
        pltpu.sync_copy(beta_hbm, beta_v)

        gammas = [gamma_v[pl.ds(j * LANES, LANES)] for j in range(NJ)]
        betas = [beta_v[pl.ds(j * LANES, LANES)] for j in range(NJ)]

        def chunk_base(k):
            kc = jnp.minimum(k, nchunk - 1)
            return row_base + kc * C

        def ifetch(k, slot):
            base = chunk_base(k)
            pltpu.make_async_copy(seq_hbm.at[pl.ds(base, C)], idx[slot],
                                  isem[slot]).start()
            pltpu.make_async_copy(lab_hbm.at[pl.ds(base, C)], labs[slot],
                                  lsem[slot]).start()

        def gstart(k, slot):
            pltpu.make_async_copy(tok_hbm.at[idx[slot]], buf[slot],
                                  gsem[slot]).start()

        def compute(k, slot):
            p0 = lax.rem(k, chunks_per_seq) * C
            buf_s = buf[slot]
            lab_s = labs[slot]

            def row_body(r, _):
                rfull = jnp.full((LANES,), r, jnp.int32)
                labelb = plsc.load_gather(lab_s, [rfull])
                segbase = labelb * DIM + lane
                pbase = (p0 + r) * DIM
                s = None
                q = None
                xs = []
                for j in range(NJ):
                    t = buf_s[r, pl.ds(j * LANES, LANES)]
                    p = pe_v[pl.ds(pbase + j * LANES, LANES)]
                    sg = plsc.load_gather(segtab_v, [segbase + j * LANES])
                    x = (t + p) + sg
                    xs.append(x)
                    s = x if s is None else s + x
                    q = x * x if q is None else q + x * x
                ssum = jnp.sum(s)
                qsum = jnp.sum(q)
                mean = jnp.full((LANES,), ssum, jnp.float32) * inv_d
                var = jnp.full((LANES,), qsum, jnp.float32) * inv_d - mean * mean
                ve = var + EPS
                seed = jnp.int32(0x5F3759DF) - (plsc.bitcast(ve, jnp.int32) >> 1)
                y = plsc.bitcast(seed, jnp.float32)
                for _ in range(3):
                    y = y * (1.5 - 0.5 * ve * y * y)
                for j in range(NJ):
                    out = (xs[j] - mean) * y * gammas[j] + betas[j]
                    buf_s[r, pl.ds(j * LANES, LANES)] = out
                return 0

            lax.fori_loop(0, C, row_body, 0, unroll=4)

        ifetch(0, 0)
        ifetch(1, 1)
        pltpu.make_async_copy(seq_hbm.at[pl.ds(row_base, C)], idx[0],
                              isem[0]).wait()
        gstart(0, 0)

        def body(k2, _):
            for par in range(2):
                k = k2 * 2 + par
                s = par
                t = 1 - par

                def wait_out():
                    pltpu.make_async_copy(
                        buf[t], out_hbm.at[pl.ds(row_base, C)], osem[t]).wait()

                if par == 1:
                    wait_out()
                else:
                    pl.when(k > 0)(wait_out)
                pltpu.make_async_copy(seq_hbm.at[pl.ds(row_base, C)], idx[t],
                                      isem[t]).wait()
                gstart(k + 1, t)
                pltpu.make_async_copy(tok_hbm.at[idx[s]], buf[s],
                                      gsem[s]).wait()
                pltpu.make_async_copy(lab_hbm.at[pl.ds(row_base, C)], labs[s],
                                      lsem[s]).wait()
                compute(k, s)
                pltpu.make_async_copy(buf[s], out_hbm.at[pl.ds(chunk_base(k), C)],
                                      osem[s]).start()
                ifetch(k + 2, s)
            return 0

        lax.fori_loop(0, nchunk // 2, body, 0)

        last = (nchunk - 1) % 2
        pltpu.make_async_copy(buf[last], out_hbm.at[pl.ds(row_base, C)],
                              osem[last]).wait()
        pltpu.make_async_copy(tok_hbm.at[idx[nchunk % 2]], buf[nchunk % 2],
                              gsem[nchunk % 2]).wait()
        pltpu.make_async_copy(seq_hbm.at[pl.ds(row_base, C)], idx[last],
                              isem[last]).wait()
        for slot in range(2):
            pltpu.make_async_copy(lab_hbm.at[pl.ds(row_base, C)], labs[slot],
                                  lsem[slot]).wait()

    return sc_kernel


def kernel(sequence, segment_label, token_table, seg_table, gamma, beta, pe):
    B, S = sequence.shape
    V = token_table.shape[0]
    seq = sequence.reshape(-1).astype(jnp.int32)
    lab = segment_label.reshape(-1).astype(jnp.int32)
    pe_s = pe[:S].reshape(-1)
    out = _build(B, S, V)(seq, lab, token_table, seg_table.reshape(-1), gamma,
                          beta, pe_s)
    return out.reshape(B, S, DIM)


# speedup vs baseline: 5.7382x; 1.4520x over previous
import functools

import jax
import jax.numpy as jnp
from jax import lax
from jax.experimental import pallas as pl
from jax.experimental.pallas import tpu as pltpu
from jax.experimental.pallas import tpu_sc as plsc

DIM = 128
NSEG = 3
EPS = 1e-5
LANES = 16
NJ = DIM // LANES
NC = 2
NS = 16
NW = NC * NS
C = 128


def _build(B, S, V):
    rows_total = B * S
    rows_per_tile = rows_total // NW
    nchunk = rows_per_tile // C
    chunks_per_seq = S // C
    inv_d = 1.0 / DIM

    mesh = plsc.VectorSubcoreMesh(core_axis_name="c", subcore_axis_name="s")

    @functools.partial(
        pl.kernel,
        mesh=mesh,
        out_type=jax.ShapeDtypeStruct((rows_total, DIM), jnp.float32),
        compiler_params=pltpu.CompilerParams(needs_layout_passes=False),
        scratch_types=[
            pltpu.VMEM((S * DIM,), jnp.float32),
            pltpu.VMEM((NSEG * DIM,), jnp.float32),
            pltpu.VMEM((DIM,), jnp.float32),
            pltpu.VMEM((DIM,), jnp.float32),
            pltpu.VMEM((C,), jnp.int32),
            pltpu.VMEM((C,), jnp.int32),
            pltpu.VMEM((C,), jnp.int32),
            pltpu.VMEM((C,), jnp.int32),
            pltpu.VMEM((C, DIM), jnp.float32),
            pltpu.VMEM((C, DIM), jnp.float32),
            pltpu.SemaphoreType.DMA,
            pltpu.SemaphoreType.DMA,
            pltpu.SemaphoreType.DMA,
            pltpu.SemaphoreType.DMA,
            pltpu.SemaphoreType.DMA,
            pltpu.SemaphoreType.DMA,
            pltpu.SemaphoreType.DMA,
            pltpu.SemaphoreType.DMA,
        ],
    )
    def sc_kernel(seq_hbm, lab_hbm, tok_hbm, segtab_hbm, gamma_hbm, beta_hbm,
                  pe_hbm, out_hbm, pe_v, segtab_v, gamma_v, beta_v, idx0, idx1,
                  lab0, lab1, buf0, buf1, isem0, isem1, lsem0, lsem1, gsem0,
                  gsem1, osem0, osem1):
        wid = lax.axis_index("s") * NC + lax.axis_index("c")
        row_base = wid * rows_per_tile
        lane = lax.iota(jnp.int32, LANES)
        idx = [idx0, idx1]
        labs = [lab0, lab1]
        buf = [buf0, buf1]
        isem = [isem0, isem1]
        lsem = [lsem0, lsem1]
        gsem = [gsem0, gsem1]
        osem = [osem0, osem1]

        pltpu.sync_copy(pe_hbm, pe_v)
        pltpu.sync_copy(segtab_hbm, segtab_v)
        pltpu.sync_copy(gamma_hbm, gamma_v)
        pltpu.sync_copy(beta_hbm, beta_v)

        gammas = [gamma_v[pl.ds(j * LANES, LANES)] for j in range(NJ)]
        betas = [beta_v[pl.ds(j * LANES, LANES)] for j in range(NJ)]

        def chunk_base(k):
            kc = jnp.minimum(k, nchunk - 1)
            return row_base + kc * C

        def ifetch(k, slot):
            base = chunk_base(k)
            pltpu.make_async_copy(seq_hbm.at[pl.ds(base, C)], idx[slot],
                                  isem[slot]).start()
            pltpu.make_async_copy(lab_hbm.at[pl.ds(base, C)], labs[slot],
                                  lsem[slot]).start()

        def gstart(k, slot):
            pltpu.make_async_copy(tok_hbm.at[idx[slot]], buf[slot],
                                  gsem[slot]).start()

        def compute(k, slot):
            p0 = lax.rem(k, chunks_per_seq) * C
            buf_s = buf[slot]
            lab_s = labs[slot]

            @plsc.parallel_loop(0, C, unroll=4)
            def row_body(r):
                rfull = jnp.full((LANES,), r, jnp.int32)
                labelb = plsc.load_gather(lab_s, [rfull])
                segbase = labelb * DIM + lane
                pbase = (p0 + r) * DIM
                s = None
                q = None
                xs = []
                for j in range(NJ):
                    t = buf_s[r, pl.ds(j * LANES, LANES)]
                    p = pe_v[pl.ds(pbase + j * LANES, LANES)]
                    sg = plsc.load_gather(segtab_v, [segbase + j * LANES])
                    x = (t + p) + sg
                    xs.append(x)
                    s = x if s is None else s + x
                    q = x * x if q is None else q + x * x
                ssum = jnp.sum(s)
                qsum = jnp.sum(q)
                mean = jnp.full((LANES,), ssum, jnp.float32) * inv_d
                var = jnp.full((LANES,), qsum, jnp.float32) * inv_d - mean * mean
                ve = var + EPS
                seed = jnp.int32(0x5F3759DF) - (plsc.bitcast(ve, jnp.int32) >> 1)
                y = plsc.bitcast(seed, jnp.float32)
                for _ in range(3):
                    y = y * (1.5 - 0.5 * ve * y * y)
                for j in range(NJ):
                    out = (xs[j] - mean) * y * gammas[j] + betas[j]
                    buf_s[r, pl.ds(j * LANES, LANES)] = out

        ifetch(0, 0)
        ifetch(1, 1)
        pltpu.make_async_copy(seq_hbm.at[pl.ds(row_base, C)], idx[0],
                              isem[0]).wait()
        gstart(0, 0)

        def body(k2, _):
            for par in range(2):
                k = k2 * 2 + par
                s = par
                t = 1 - par

                def wait_out():
                    pltpu.make_async_copy(
                        buf[t], out_hbm.at[pl.ds(row_base, C)], osem[t]).wait()

                if par == 1:
                    wait_out()
                else:
                    pl.when(k > 0)(wait_out)
                pltpu.make_async_copy(seq_hbm.at[pl.ds(row_base, C)], idx[t],
                                      isem[t]).wait()
                gstart(k + 1, t)
                pltpu.make_async_copy(tok_hbm.at[idx[s]], buf[s],
                                      gsem[s]).wait()
                pltpu.make_async_copy(lab_hbm.at[pl.ds(row_base, C)], labs[s],
                                      lsem[s]).wait()
                compute(k, s)
                pltpu.make_async_copy(buf[s], out_hbm.at[pl.ds(chunk_base(k), C)],
                                      osem[s]).start()
                ifetch(k + 2, s)
            return 0

        lax.fori_loop(0, nchunk // 2, body, 0)

        last = (nchunk - 1) % 2
        pltpu.make_async_copy(buf[last], out_hbm.at[pl.ds(row_base, C)],
                              osem[last]).wait()
        pltpu.make_async_copy(tok_hbm.at[idx[nchunk % 2]], buf[nchunk % 2],
                              gsem[nchunk % 2]).wait()
        pltpu.make_async_copy(seq_hbm.at[pl.ds(row_base, C)], idx[last],
                              isem[last]).wait()
        for slot in range(2):
            pltpu.make_async_copy(lab_hbm.at[pl.ds(row_base, C)], labs[slot],
                                  lsem[slot]).wait()

    return sc_kernel


def kernel(sequence, segment_label, token_table, seg_table, gamma, beta, pe):
    B, S = sequence.shape
    V = token_table.shape[0]
    seq = sequence.reshape(-1).astype(jnp.int32)
    lab = segment_label.reshape(-1).astype(jnp.int32)
    pe_s = pe[:S].reshape(-1)
    out = _build(B, S, V)(seq, lab, token_table, seg_table.reshape(-1), gamma,
                          beta, pe_s)
    return out.reshape(B, S, DIM)


# speedup vs baseline: 6.0064x; 1.0467x over previous
import functools

import jax
import jax.numpy as jnp
from jax import lax
from jax.experimental import pallas as pl
from jax.experimental.pallas import tpu as pltpu
from jax.experimental.pallas import tpu_sc as plsc

DIM = 128
NSEG = 3
EPS = 1e-5
LANES = 16
NJ = DIM // LANES
NC = 2
NS = 16
NW = NC * NS
C = 128


def _build(B, S, V):
    rows_total = B * S
    rows_per_tile = rows_total // NW
    nchunk = rows_per_tile // C
    chunks_per_seq = S // C
    inv_d = 1.0 / DIM

    mesh = plsc.VectorSubcoreMesh(core_axis_name="c", subcore_axis_name="s")

    @functools.partial(
        pl.kernel,
        mesh=mesh,
        out_type=jax.ShapeDtypeStruct((rows_total, DIM), jnp.float32),
        compiler_params=pltpu.CompilerParams(needs_layout_passes=False),
        scratch_types=[
            pltpu.VMEM((S * DIM,), jnp.float32),
            pltpu.VMEM((NSEG * DIM,), jnp.float32),
            pltpu.VMEM((DIM,), jnp.float32),
            pltpu.VMEM((DIM,), jnp.float32),
            pltpu.VMEM((C,), jnp.int32),
            pltpu.VMEM((C,), jnp.int32),
            pltpu.VMEM((C,), jnp.int32),
            pltpu.VMEM((C,), jnp.int32),
            pltpu.VMEM((C, DIM), jnp.float32),
            pltpu.VMEM((C, DIM), jnp.float32),
            pltpu.SemaphoreType.DMA,
            pltpu.SemaphoreType.DMA,
            pltpu.SemaphoreType.DMA,
            pltpu.SemaphoreType.DMA,
            pltpu.SemaphoreType.DMA,
            pltpu.SemaphoreType.DMA,
            pltpu.SemaphoreType.DMA,
            pltpu.SemaphoreType.DMA,
        ],
    )
    def sc_kernel(seq_hbm, lab_hbm, tok_hbm, segtab_hbm, gamma_hbm, beta_hbm,
                  pe_hbm, out_hbm, pe_v, segtab_v, gamma_v, beta_v, idx0, idx1,
                  lab0, lab1, buf0, buf1, isem0, isem1, lsem0, lsem1, gsem0,
                  gsem1, osem0, osem1):
        wid = lax.axis_index("s") * NC + lax.axis_index("c")
        row_base = wid * rows_per_tile
        lane = lax.iota(jnp.int32, LANES)
        idx = [idx0, idx1]
        labs = [lab0, lab1]
        buf = [buf0, buf1]
        isem = [isem0, isem1]
        lsem = [lsem0, lsem1]
        gsem = [gsem0, gsem1]
        osem = [osem0, osem1]

        pltpu.sync_copy(pe_hbm, pe_v)
        pltpu.sync_copy(segtab_hbm, segtab_v)
        pltpu.sync_copy(gamma_hbm, gamma_v)
        pltpu.sync_copy(beta_hbm, beta_v)

        gammas = [gamma_v[pl.ds(j * LANES, LANES)] for j in range(NJ)]
        betas = [beta_v[pl.ds(j * LANES, LANES)] for j in range(NJ)]

        def chunk_base(k):
            kc = jnp.minimum(k, nchunk - 1)
            return row_base + kc * C

        def ifetch(k, slot):
            base = chunk_base(k)
            pltpu.make_async_copy(seq_hbm.at[pl.ds(base, C)], idx[slot],
                                  isem[slot]).start()
            pltpu.make_async_copy(lab_hbm.at[pl.ds(base, C)], labs[slot],
                                  lsem[slot]).start()

        def gstart(k, slot):
            pltpu.make_async_copy(tok_hbm.at[idx[slot]], buf[slot],
                                  gsem[slot]).start()

        def compute(k, slot):
            p0 = lax.rem(k, chunks_per_seq) * C
            buf_s = buf[slot]
            lab_s = labs[slot]

            @plsc.parallel_loop(0, C, unroll=8)
            def row_body(r):
                rfull = jnp.full((LANES,), r, jnp.int32)
                labelb = plsc.load_gather(lab_s, [rfull])
                segbase = labelb * DIM + lane
                pbase = (p0 + r) * DIM
                s = None
                q = None
                xs = []
                for j in range(NJ):
                    t = buf_s[r, pl.ds(j * LANES, LANES)]
                    p = pe_v[pl.ds(pbase + j * LANES, LANES)]
                    sg = plsc.load_gather(segtab_v, [segbase + j * LANES])
                    x = (t + p) + sg
                    xs.append(x)
                    s = x if s is None else s + x
                    q = x * x if q is None else q + x * x
                ssum = jnp.sum(s)
                qsum = jnp.sum(q)
                mean = jnp.full((LANES,), ssum, jnp.float32) * inv_d
                var = jnp.full((LANES,), qsum, jnp.float32) * inv_d - mean * mean
                ve = var + EPS
                seed = jnp.int32(0x5F3759DF) - (plsc.bitcast(ve, jnp.int32) >> 1)
                y = plsc.bitcast(seed, jnp.float32)
                for _ in range(2):
                    y = y * (1.5 - 0.5 * ve * y * y)
                for j in range(NJ):
                    out = (xs[j] - mean) * y * gammas[j] + betas[j]
                    buf_s[r, pl.ds(j * LANES, LANES)] = out

        ifetch(0, 0)
        ifetch(1, 1)
        pltpu.make_async_copy(seq_hbm.at[pl.ds(row_base, C)], idx[0],
                              isem[0]).wait()
        gstart(0, 0)

        def body(k2, _):
            for par in range(2):
                k = k2 * 2 + par
                s = par
                t = 1 - par

                def wait_out():
                    pltpu.make_async_copy(
                        buf[t], out_hbm.at[pl.ds(row_base, C)], osem[t]).wait()

                if par == 1:
                    wait_out()
                else:
                    pl.when(k > 0)(wait_out)
                pltpu.make_async_copy(seq_hbm.at[pl.ds(row_base, C)], idx[t],
                                      isem[t]).wait()
                gstart(k + 1, t)
                pltpu.make_async_copy(tok_hbm.at[idx[s]], buf[s],
                                      gsem[s]).wait()
                pltpu.make_async_copy(lab_hbm.at[pl.ds(row_base, C)], labs[s],
                                      lsem[s]).wait()
                compute(k, s)
                pltpu.make_async_copy(buf[s], out_hbm.at[pl.ds(chunk_base(k), C)],
                                      osem[s]).start()
                ifetch(k + 2, s)
            return 0

        lax.fori_loop(0, nchunk // 2, body, 0)

        last = (nchunk - 1) % 2
        pltpu.make_async_copy(buf[last], out_hbm.at[pl.ds(row_base, C)],
                              osem[last]).wait()
        pltpu.make_async_copy(tok_hbm.at[idx[nchunk % 2]], buf[nchunk % 2],
                              gsem[nchunk % 2]).wait()
        pltpu.make_async_copy(seq_hbm.at[pl.ds(row_base, C)], idx[last],
                              isem[last]).wait()
        for slot in range(2):
            pltpu.make_async_copy(lab_hbm.at[pl.ds(row_base, C)], labs[slot],
                                  lsem[slot]).wait()

    return sc_kernel


def kernel(sequence, segment_label, token_table, seg_table, gamma, beta, pe):
    B, S = sequence.shape
    V = token_table.shape[0]
    seq = sequence.reshape(-1).astype(jnp.int32)
    lab = segment_label.reshape(-1).astype(jnp.int32)
    pe_s = pe[:S].reshape(-1)
    out = _build(B, S, V)(seq, lab, token_table, seg_table.reshape(-1), gamma,
                          beta, pe_s)
    return out.reshape(B, S, DIM)
